# baseline (speedup 1.0000x reference)
import functools
import math

import jax
import jax.numpy as jnp
from jax import lax
from jax.experimental import pallas as pl
from jax.experimental.pallas import tpu as pltpu
from jax.experimental.pallas import tpu_sc as plsc


def _mm(a, b):
    return lax.dot_general(a, b, (((1,), (0,)), ((), ())),
                           preferred_element_type=jnp.float32)


def _mm_t(a, b):
    return lax.dot_general(a, b, (((1,), (1,)), ((), ())),
                           preferred_element_type=jnp.float32)


def _bf16_bits(x):
    xr = x.astype(jnp.bfloat16).astype(jnp.float32)
    return lax.bitcast_convert_type(xr, jnp.uint32)


def _prep(uf, um, isf_t, imem_t):
    n, ud = uf.shape
    fd = isf_t.shape[1]
    md = imem_t.shape[1]
    hd = ud // 2
    blk = 2000

    def body(uf_ref, um_ref, isf_ref, imem_ref, u_ref, it_ref):
        a = uf_ref[...] + um_ref[...]
        lo = _bf16_bits(a[:, :hd]) >> 16
        hi = _bf16_bits(a[:, hd:]) & jnp.uint32(0xFFFF0000)
        u_ref[...] = lax.bitcast_convert_type(lo | hi, jnp.int32)
        ilo = _bf16_bits(isf_ref[...]) >> 16
        ihi = _bf16_bits(imem_ref[...]) & jnp.uint32(0xFFFF0000)
        it_ref[...] = lax.bitcast_convert_type(ilo | ihi, jnp.int32)

    assert n % blk == 0 and fd == md == hd
    return pl.pallas_call(
        body,
        grid=(n // blk,),
        in_specs=[pl.BlockSpec((blk, ud), lambda i: (i, 0)),
                  pl.BlockSpec((blk, ud), lambda i: (i, 0)),
                  pl.BlockSpec((blk, fd), lambda i: (i, 0)),
                  pl.BlockSpec((blk, md), lambda i: (i, 0))],
        out_specs=[pl.BlockSpec((blk, hd), lambda i: (i, 0)),
                   pl.BlockSpec((blk, hd), lambda i: (i, 0))],
        out_shape=[jax.ShapeDtypeStruct((n, hd), jnp.int32),
                   jax.ShapeDtypeStruct((n, hd), jnp.int32)],
    )(uf, um, isf_t, imem_t)


def _sc_gather(U, IT, vidx, iidx, inv):
    info = plsc.get_sparse_core_info()
    NC, NS = info.num_cores, info.num_subcores
    NW = NC * NS
    E = vidx.shape[0]
    Bq = inv.shape[0]
    UD = U.shape[1]
    KD2 = IT.shape[1]
    C = 128
    per_w = E // NW
    n_chunks = per_w // C
    src_per_w = Bq // NW
    assert per_w * NW == E and n_chunks * C == per_w and n_chunks % 2 == 0
    assert src_per_w * NW == Bq and src_per_w <= C

    mesh = plsc.VectorSubcoreMesh(core_axis_name="c", subcore_axis_name="s")

    @functools.partial(
        pl.kernel,
        mesh=mesh,
        out_type=[jax.ShapeDtypeStruct((E, UD), jnp.int32),
                  jax.ShapeDtypeStruct((E, KD2), jnp.int32),
                  jax.ShapeDtypeStruct((Bq, UD), jnp.int32)],
        scratch_types=[pltpu.VMEM((per_w,), jnp.int32),
                       pltpu.VMEM((per_w,), jnp.int32),
                       pltpu.VMEM((C, UD), jnp.int32),
                       pltpu.VMEM((C, KD2), jnp.int32),
                       pltpu.VMEM((C, UD), jnp.int32),
                       pltpu.VMEM((C, KD2), jnp.int32),
                       pltpu.SemaphoreType.DMA,
                       pltpu.SemaphoreType.DMA],
    )
    def gather_k(u_hbm, it_hbm, vidx_hbm, iidx_hbm, inv_hbm,
                 nb_out, it_out, src_out,
                 idx_u, idx_i, nb_a, it_a, nb_b, it_b, sem_a, sem_b):
        wid = lax.axis_index("s") * NC + lax.axis_index("c")
        base0 = wid * per_w

        pltpu.sync_copy(vidx_hbm.at[pl.ds(base0, per_w)], idx_u)
        pltpu.sync_copy(iidx_hbm.at[pl.ds(base0, per_w)], idx_i)

        def startg(j, nb_v, it_v, sem):
            off = j * C
            c1 = pltpu.async_copy(u_hbm.at[idx_u.at[pl.ds(off, C)]], nb_v, sem)
            c2 = pltpu.async_copy(it_hbm.at[idx_i.at[pl.ds(off, C)]], it_v, sem)
            return c1, c2

        pa = startg(0, nb_a, it_a, sem_a)
        pb = startg(1, nb_b, it_b, sem_b)

        def body(t, carry):
            j0 = 2 * t
            base = base0 + j0 * C
            pltpu.make_async_copy(u_hbm.at[idx_u.at[pl.ds(0, C)]], nb_a,
                                  sem_a).wait()
            pltpu.make_async_copy(it_hbm.at[idx_i.at[pl.ds(0, C)]], it_a,
                                  sem_a).wait()
            pltpu.sync_copy(nb_a, nb_out.at[pl.ds(base, C)])
            pltpu.sync_copy(it_a, it_out.at[pl.ds(base, C)])

            @pl.when(t + 1 < n_chunks // 2)
            def _():
                startg(j0 + 2, nb_a, it_a, sem_a)

            pltpu.make_async_copy(u_hbm.at[idx_u.at[pl.ds(0, C)]], nb_b,
                                  sem_b).wait()
            pltpu.make_async_copy(it_hbm.at[idx_i.at[pl.ds(0, C)]], it_b,
                                  sem_b).wait()
            pltpu.sync_copy(nb_b, nb_out.at[pl.ds(base + C, C)])
            pltpu.sync_copy(it_b, it_out.at[pl.ds(base + C, C)])

            @pl.when(t + 1 < n_chunks // 2)
            def _():
                startg(j0 + 3, nb_b, it_b, sem_b)

            return carry

        lax.fori_loop(0, n_chunks // 2, body, 0)

        sbase = wid * src_per_w
        pltpu.sync_copy(inv_hbm.at[pl.ds(sbase, src_per_w)],
                        idx_u.at[pl.ds(0, src_per_w)])
        pltpu.async_copy(u_hbm.at[idx_u.at[pl.ds(0, src_per_w)]],
                         nb_a.at[pl.ds(0, src_per_w)], sem_a).wait()
        pltpu.sync_copy(nb_a.at[pl.ds(0, src_per_w)],
                        src_out.at[pl.ds(sbase, src_per_w)])

    return gather_k(U, IT, vidx, iidx, inv)


def _unpack(p):
    lo = lax.bitcast_convert_type(jnp.left_shift(p, 16), jnp.float32)
    hi = lax.bitcast_convert_type(
        jnp.bitwise_and(p, jnp.int32(-65536)), jnp.float32)
    return lo, hi


def _fast_cos(x):
    n = jnp.round(x * jnp.float32(0.15915494309189535))
    r = x - n * jnp.float32(6.2831854820251465)
    r = r - n * jnp.float32(-1.7484556000744487e-07)
    r2 = r * r
    p = jnp.float32(-1.147074559772972e-11)
    p = p * r2 + jnp.float32(2.08767569878681e-09)
    p = p * r2 + jnp.float32(-2.7557319223985893e-07)
    p = p * r2 + jnp.float32(2.480158730158730e-05)
    p = p * r2 + jnp.float32(-1.3888888888888889e-03)
    p = p * r2 + jnp.float32(4.1666666666666664e-02)
    p = p * r2 + jnp.float32(-0.5)
    return p * r2 + jnp.float32(1.0)


def _main_body(nb_ref, it_ref, src_ref, ts_ref, ets_ref, vidx_ref,
               tw_ref, tb_ref, wq_ref, bq_ref, wk_ref, bk_ref, wv_ref, bv_ref,
               wo_ref, bo_ref, w1_ref, b1_ref, w2_ref, b2_ref, o_ref):
    HD = src_ref.shape[1]
    UD = 2 * HD
    TD = tw_ref.shape[1]
    QD = wq_ref.shape[0]
    H = 2
    DH = QD // H

    src_lo, src_hi = _unpack(src_ref[...])
    tw = tw_ref[...]
    tb = tb_ref[...]

    st = jnp.cos(tb)
    wq = wq_ref[...]
    q = (_mm_t(src_lo, wq[:, :HD]) + _mm_t(src_hi, wq[:, HD:UD])
         + _mm_t(st, wq[:, UD:]) + bq_ref[...])
    q = q * jnp.float32(1.0 / math.sqrt(DH))

    wk = wk_ref[...]
    g0 = _mm(q[:, :DH], wk[:DH, :])
    g1 = _mm(q[:, DH:], wk[DH:, :])
    bk = bk_ref[...]
    sb0 = jnp.sum(q[:, :DH] * bk[:, :DH], axis=-1, keepdims=True)
    sb1 = jnp.sum(q[:, DH:] * bk[:, DH:], axis=-1, keepdims=True)

    deltas = ts_ref[...] - ets_ref[...]
    TDr = tw.reshape(1, 1, TD)
    TBr = tb.reshape(1, 1, TD)
    ete = _fast_cos(deltas[:, :, None] * TDr + TBr)

    nb_lo, nb_hi = _unpack(nb_ref[...])
    it_lo, it_hi = _unpack(it_ref[...])

    mask = vidx_ref[...] == 0
    neg = jnp.float32(-1e10)

    o1 = UD
    o2 = UD + TD
    o3 = UD + TD + HD

    def head_attn(g, sb):
        prod = (nb_lo * g[:, None, :HD] + nb_hi * g[:, None, HD:o1]
                + ete * g[:, None, o1:o2] + it_lo * g[:, None, o2:o3]
                + it_hi * g[:, None, o3:])
        s = jnp.sum(prod, axis=-1) + sb
        s = jnp.where(mask, neg, s)
        m = jnp.max(s, axis=-1, keepdims=True)
        e = jnp.exp(s - m)
        return e / jnp.sum(e, axis=-1, keepdims=True)

    a0 = head_attn(g0, sb0)
    a1 = head_attn(g1, sb1)

    wv = wv_ref[...]
    bv = bv_ref[...]

    def head_ctx(a, lo):
        w = wv[lo:lo + DH, :]
        a3 = a[:, :, None]
        pooled = jnp.concatenate(
            [jnp.sum(a3 * nb_lo, axis=1), jnp.sum(a3 * nb_hi, axis=1),
             jnp.sum(a3 * ete, axis=1), jnp.sum(a3 * it_lo, axis=1),
             jnp.sum(a3 * it_hi, axis=1)], axis=-1)
        return _mm_t(pooled, w) + bv[:, lo:lo + DH]

    ctx = jnp.concatenate([head_ctx(a0, 0), head_ctx(a1, DH)], axis=-1)
    attn_out = _mm_t(ctx, wo_ref[...]) + bo_ref[...]
    allm = jnp.all(mask, axis=-1, keepdims=True)
    attn_out = jnp.where(allm, jnp.float32(0.0), attn_out)

    w1 = w1_ref[...]
    hid = jnp.maximum(
        _mm_t(attn_out, w1[:, :QD]) + _mm_t(src_lo, w1[:, QD:QD + HD])
        + _mm_t(src_hi, w1[:, QD + HD:]) + b1_ref[...],
        jnp.float32(0.0))
    o_ref[...] = _mm_t(hid, w2_ref[...]) + b2_ref[...]


def _main(nb, it, src, ts, ets, vidx,
          tw, tb, Wq, bq, Wk, bk, Wv, bv, Wo, bo, W1, b1, W2, b2):
    B, NN, HD = nb.shape
    KD2 = it.shape[2]
    OD = W2.shape[0]
    BB = 256
    assert B % BB == 0
    grid = (B // BB,)

    def blk(shape):
        nd = len(shape)
        return pl.BlockSpec(shape, lambda i, _n=nd: (i,) + (0,) * (_n - 1))

    def full(a):
        nd = a.ndim
        return pl.BlockSpec(a.shape, lambda i, _n=nd: (0,) * _n)

    weights = (tw, tb, Wq, bq, Wk, bk, Wv, bv, Wo, bo, W1, b1, W2, b2)
    in_specs = [blk((BB, NN, HD)), blk((BB, NN, KD2)),
                blk((BB, HD)), blk((BB, 1)), blk((BB, NN)), blk((BB, NN))]
    in_specs += [full(w) for w in weights]

    return pl.pallas_call(
        _main_body,
        grid=grid,
        in_specs=in_specs,
        out_specs=blk((BB, OD)),
        out_shape=jax.ShapeDtypeStruct((B, OD), jnp.float32),
    )(nb, it, src, ts, ets, vidx, *weights)


def kernel(inviters_idxs, timestamps, num_layers, num_neighbors,
           neighbors_voters_idxs, neighbors_items_idxs,
           neighbors_edges_timestamps, users_features, users_memory,
           items_static_features, items_memory, time_w, time_b,
           Wq, bq, Wk, bk, Wv, bv, Wo, bo, W1, b1, W2, b2):
    B = inviters_idxs.shape[0]
    NN = neighbors_voters_idxs.shape[1]

    U, IT = _prep(users_features, users_memory,
                  items_static_features, items_memory)

    vidx32 = neighbors_voters_idxs.astype(jnp.int32)
    iidx32 = neighbors_items_idxs.astype(jnp.int32)
    inv = inviters_idxs.astype(jnp.int32)
    ts2 = timestamps.reshape(B, 1)
    wargs = (time_w.reshape(1, -1), time_b.reshape(1, -1),
             Wq, bq.reshape(1, -1), Wk, bk.reshape(1, -1),
             Wv, bv.reshape(1, -1), Wo, bo.reshape(1, -1),
             W1, b1.reshape(1, -1), W2, b2.reshape(1, -1))

    nparts = 4
    h = B // nparts
    outs = []
    for p in range(nparts):
        sl = slice(p * h, (p + 1) * h)
        nbf, itf, srcf = _sc_gather(
            U, IT, vidx32[sl].reshape(-1), iidx32[sl].reshape(-1), inv[sl])
        outs.append(_main(
            nbf.reshape(h, NN, -1), itf.reshape(h, NN, -1), srcf,
            ts2[sl], neighbors_edges_timestamps[sl], vidx32[sl], *wargs))
    return jnp.concatenate(outs, axis=0)


# speedup vs baseline: 1.0022x; 1.0022x over previous
import functools
import math

import jax
import jax.numpy as jnp
from jax import lax
from jax.experimental import pallas as pl
from jax.experimental.pallas import tpu as pltpu
from jax.experimental.pallas import tpu_sc as plsc


def _mm(a, b):
    return lax.dot_general(a, b, (((1,), (0,)), ((), ())),
                           preferred_element_type=jnp.float32)


def _mm_t(a, b):
    return lax.dot_general(a, b, (((1,), (1,)), ((), ())),
                           preferred_element_type=jnp.float32)


def _bf16_bits(x):
    xr = x.astype(jnp.bfloat16).astype(jnp.float32)
    return lax.bitcast_convert_type(xr, jnp.uint32)


def _prep(uf, um, isf_t, imem_t):
    n, ud = uf.shape
    fd = isf_t.shape[1]
    md = imem_t.shape[1]
    hd = ud // 2
    blk = 2000

    def body(uf_ref, um_ref, isf_ref, imem_ref, u_ref, it_ref):
        a = uf_ref[...] + um_ref[...]
        lo = _bf16_bits(a[:, :hd]) >> 16
        hi = _bf16_bits(a[:, hd:]) & jnp.uint32(0xFFFF0000)
        u_ref[...] = lax.bitcast_convert_type(lo | hi, jnp.int32)
        ilo = _bf16_bits(isf_ref[...]) >> 16
        ihi = _bf16_bits(imem_ref[...]) & jnp.uint32(0xFFFF0000)
        it_ref[...] = lax.bitcast_convert_type(ilo | ihi, jnp.int32)

    assert n % blk == 0 and fd == md == hd
    return pl.pallas_call(
        body,
        grid=(n // blk,),
        in_specs=[pl.BlockSpec((blk, ud), lambda i: (i, 0)),
                  pl.BlockSpec((blk, ud), lambda i: (i, 0)),
                  pl.BlockSpec((blk, fd), lambda i: (i, 0)),
                  pl.BlockSpec((blk, md), lambda i: (i, 0))],
        out_specs=[pl.BlockSpec((blk, hd), lambda i: (i, 0)),
                   pl.BlockSpec((blk, hd), lambda i: (i, 0))],
        out_shape=[jax.ShapeDtypeStruct((n, hd), jnp.int32),
                   jax.ShapeDtypeStruct((n, hd), jnp.int32)],
    )(uf, um, isf_t, imem_t)


def _sc_gather(U, IT, vidx, iidx, inv):
    info = plsc.get_sparse_core_info()
    NC, NS = info.num_cores, info.num_subcores
    NW = NC * NS
    E = vidx.shape[0]
    Bq = inv.shape[0]
    UD = U.shape[1]
    KD2 = IT.shape[1]
    C = 128
    per_w = E // NW
    n_chunks = per_w // C
    src_per_w = Bq // NW
    assert per_w * NW == E and n_chunks * C == per_w and n_chunks % 2 == 0
    assert src_per_w * NW == Bq and src_per_w <= C

    mesh = plsc.VectorSubcoreMesh(core_axis_name="c", subcore_axis_name="s")

    @functools.partial(
        pl.kernel,
        mesh=mesh,
        out_type=[jax.ShapeDtypeStruct((E, UD), jnp.int32),
                  jax.ShapeDtypeStruct((E, KD2), jnp.int32),
                  jax.ShapeDtypeStruct((Bq, UD), jnp.int32)],
        scratch_types=[pltpu.VMEM((per_w,), jnp.int32),
                       pltpu.VMEM((per_w,), jnp.int32),
                       pltpu.VMEM((C, UD), jnp.int32),
                       pltpu.VMEM((C, KD2), jnp.int32),
                       pltpu.VMEM((C, UD), jnp.int32),
                       pltpu.VMEM((C, KD2), jnp.int32),
                       pltpu.SemaphoreType.DMA,
                       pltpu.SemaphoreType.DMA],
    )
    def gather_k(u_hbm, it_hbm, vidx_hbm, iidx_hbm, inv_hbm,
                 nb_out, it_out, src_out,
                 idx_u, idx_i, nb_a, it_a, nb_b, it_b, sem_a, sem_b):
        wid = lax.axis_index("s") * NC + lax.axis_index("c")
        base0 = wid * per_w

        pltpu.sync_copy(vidx_hbm.at[pl.ds(base0, per_w)], idx_u)
        pltpu.sync_copy(iidx_hbm.at[pl.ds(base0, per_w)], idx_i)

        def startg(j, nb_v, it_v, sem):
            off = j * C
            c1 = pltpu.async_copy(u_hbm.at[idx_u.at[pl.ds(off, C)]], nb_v, sem)
            c2 = pltpu.async_copy(it_hbm.at[idx_i.at[pl.ds(off, C)]], it_v, sem)
            return c1, c2

        pa = startg(0, nb_a, it_a, sem_a)
        pb = startg(1, nb_b, it_b, sem_b)

        def body(t, carry):
            j0 = 2 * t
            base = base0 + j0 * C
            pltpu.make_async_copy(u_hbm.at[idx_u.at[pl.ds(0, C)]], nb_a,
                                  sem_a).wait()
            pltpu.make_async_copy(it_hbm.at[idx_i.at[pl.ds(0, C)]], it_a,
                                  sem_a).wait()
            pltpu.sync_copy(nb_a, nb_out.at[pl.ds(base, C)])
            pltpu.sync_copy(it_a, it_out.at[pl.ds(base, C)])

            @pl.when(t + 1 < n_chunks // 2)
            def _():
                startg(j0 + 2, nb_a, it_a, sem_a)

            pltpu.make_async_copy(u_hbm.at[idx_u.at[pl.ds(0, C)]], nb_b,
                                  sem_b).wait()
            pltpu.make_async_copy(it_hbm.at[idx_i.at[pl.ds(0, C)]], it_b,
                                  sem_b).wait()
            pltpu.sync_copy(nb_b, nb_out.at[pl.ds(base + C, C)])
            pltpu.sync_copy(it_b, it_out.at[pl.ds(base + C, C)])

            @pl.when(t + 1 < n_chunks // 2)
            def _():
                startg(j0 + 3, nb_b, it_b, sem_b)

            return carry

        lax.fori_loop(0, n_chunks // 2, body, 0)

        sbase = wid * src_per_w
        pltpu.sync_copy(inv_hbm.at[pl.ds(sbase, src_per_w)],
                        idx_u.at[pl.ds(0, src_per_w)])
        pltpu.async_copy(u_hbm.at[idx_u.at[pl.ds(0, src_per_w)]],
                         nb_a.at[pl.ds(0, src_per_w)], sem_a).wait()
        pltpu.sync_copy(nb_a.at[pl.ds(0, src_per_w)],
                        src_out.at[pl.ds(sbase, src_per_w)])

    return gather_k(U, IT, vidx, iidx, inv)


def _unpack(p):
    lo = lax.bitcast_convert_type(jnp.left_shift(p, 16), jnp.float32)
    hi = lax.bitcast_convert_type(
        jnp.bitwise_and(p, jnp.int32(-65536)), jnp.float32)
    return lo, hi


def _fast_cos(x):
    n = jnp.round(x * jnp.float32(0.15915494309189535))
    r = x - n * jnp.float32(6.2831854820251465)
    r = r - n * jnp.float32(-1.7484556000744487e-07)
    r2 = r * r
    p = jnp.float32(-1.147074559772972e-11)
    p = p * r2 + jnp.float32(2.08767569878681e-09)
    p = p * r2 + jnp.float32(-2.7557319223985893e-07)
    p = p * r2 + jnp.float32(2.480158730158730e-05)
    p = p * r2 + jnp.float32(-1.3888888888888889e-03)
    p = p * r2 + jnp.float32(4.1666666666666664e-02)
    p = p * r2 + jnp.float32(-0.5)
    return p * r2 + jnp.float32(1.0)


def _main_body(nb_ref, it_ref, src_ref, ts_ref, ets_ref, vidx_ref,
               tw_ref, tb_ref, wq_ref, bq_ref, wk_ref, bk_ref, wv_ref, bv_ref,
               wo_ref, bo_ref, w1_ref, b1_ref, w2_ref, b2_ref, o_ref):
    HD = src_ref.shape[1]
    UD = 2 * HD
    TD = tw_ref.shape[1]
    QD = wq_ref.shape[0]
    H = 2
    DH = QD // H

    src_lo, src_hi = _unpack(src_ref[...])
    tw = tw_ref[...]
    tb = tb_ref[...]

    st = jnp.cos(tb)
    wq = wq_ref[...]
    q = (_mm_t(src_lo, wq[:, :HD]) + _mm_t(src_hi, wq[:, HD:UD])
         + _mm_t(st, wq[:, UD:]) + bq_ref[...])
    q = q * jnp.float32(1.0 / math.sqrt(DH))

    wk = wk_ref[...]
    g0 = _mm(q[:, :DH], wk[:DH, :])
    g1 = _mm(q[:, DH:], wk[DH:, :])
    bk = bk_ref[...]
    sb0 = jnp.sum(q[:, :DH] * bk[:, :DH], axis=-1, keepdims=True)
    sb1 = jnp.sum(q[:, DH:] * bk[:, DH:], axis=-1, keepdims=True)

    deltas = ts_ref[...] - ets_ref[...]
    TDr = tw.reshape(1, 1, TD)
    TBr = tb.reshape(1, 1, TD)
    ete = _fast_cos(deltas[:, :, None] * TDr + TBr)

    nb_lo, nb_hi = _unpack(nb_ref[...])
    it_lo, it_hi = _unpack(it_ref[...])

    mask = vidx_ref[...] == 0
    neg = jnp.float32(-1e10)

    o1 = UD
    o2 = UD + TD
    o3 = UD + TD + HD

    def head_attn(g, sb):
        prod = (nb_lo * g[:, None, :HD] + nb_hi * g[:, None, HD:o1]
                + ete * g[:, None, o1:o2] + it_lo * g[:, None, o2:o3]
                + it_hi * g[:, None, o3:])
        s = jnp.sum(prod, axis=-1) + sb
        s = jnp.where(mask, neg, s)
        m = jnp.max(s, axis=-1, keepdims=True)
        e = jnp.exp(s - m)
        return e / jnp.sum(e, axis=-1, keepdims=True)

    a0 = head_attn(g0, sb0)
    a1 = head_attn(g1, sb1)

    wv = wv_ref[...]
    bv = bv_ref[...]

    def head_ctx(a, lo):
        w = wv[lo:lo + DH, :]
        a3 = a[:, :, None]
        pooled = jnp.concatenate(
            [jnp.sum(a3 * nb_lo, axis=1), jnp.sum(a3 * nb_hi, axis=1),
             jnp.sum(a3 * ete, axis=1), jnp.sum(a3 * it_lo, axis=1),
             jnp.sum(a3 * it_hi, axis=1)], axis=-1)
        return _mm_t(pooled, w) + bv[:, lo:lo + DH]

    ctx = jnp.concatenate([head_ctx(a0, 0), head_ctx(a1, DH)], axis=-1)
    attn_out = _mm_t(ctx, wo_ref[...]) + bo_ref[...]
    allm = jnp.all(mask, axis=-1, keepdims=True)
    attn_out = jnp.where(allm, jnp.float32(0.0), attn_out)

    w1 = w1_ref[...]
    hid = jnp.maximum(
        _mm_t(attn_out, w1[:, :QD]) + _mm_t(src_lo, w1[:, QD:QD + HD])
        + _mm_t(src_hi, w1[:, QD + HD:]) + b1_ref[...],
        jnp.float32(0.0))
    o_ref[...] = _mm_t(hid, w2_ref[...]) + b2_ref[...]


def _main(nb, it, src, ts, ets, vidx,
          tw, tb, Wq, bq, Wk, bk, Wv, bv, Wo, bo, W1, b1, W2, b2):
    B, NN, HD = nb.shape
    KD2 = it.shape[2]
    OD = W2.shape[0]
    BB = 256
    assert B % BB == 0
    grid = (B // BB,)

    def blk(shape):
        nd = len(shape)
        return pl.BlockSpec(shape, lambda i, _n=nd: (i,) + (0,) * (_n - 1))

    def full(a):
        nd = a.ndim
        return pl.BlockSpec(a.shape, lambda i, _n=nd: (0,) * _n)

    weights = (tw, tb, Wq, bq, Wk, bk, Wv, bv, Wo, bo, W1, b1, W2, b2)
    in_specs = [blk((BB, NN, HD)), blk((BB, NN, KD2)),
                blk((BB, HD)), blk((BB, 1)), blk((BB, NN)), blk((BB, NN))]
    in_specs += [full(w) for w in weights]

    return pl.pallas_call(
        _main_body,
        grid=grid,
        in_specs=in_specs,
        out_specs=blk((BB, OD)),
        out_shape=jax.ShapeDtypeStruct((B, OD), jnp.float32),
    )(nb, it, src, ts, ets, vidx, *weights)


def kernel(inviters_idxs, timestamps, num_layers, num_neighbors,
           neighbors_voters_idxs, neighbors_items_idxs,
           neighbors_edges_timestamps, users_features, users_memory,
           items_static_features, items_memory, time_w, time_b,
           Wq, bq, Wk, bk, Wv, bv, Wo, bo, W1, b1, W2, b2):
    B = inviters_idxs.shape[0]
    NN = neighbors_voters_idxs.shape[1]

    U, IT = _prep(users_features, users_memory,
                  items_static_features, items_memory)

    vidx32 = neighbors_voters_idxs.astype(jnp.int32)
    iidx32 = neighbors_items_idxs.astype(jnp.int32)
    inv = inviters_idxs.astype(jnp.int32)
    ts2 = timestamps.reshape(B, 1)
    wargs = (time_w.reshape(1, -1), time_b.reshape(1, -1),
             Wq, bq.reshape(1, -1), Wk, bk.reshape(1, -1),
             Wv, bv.reshape(1, -1), Wo, bo.reshape(1, -1),
             W1, b1.reshape(1, -1), W2, b2.reshape(1, -1))

    nparts = 2
    h = B // nparts
    outs = []
    for p in range(nparts):
        sl = slice(p * h, (p + 1) * h)
        nbf, itf, srcf = _sc_gather(
            U, IT, vidx32[sl].reshape(-1), iidx32[sl].reshape(-1), inv[sl])
        outs.append(_main(
            nbf.reshape(h, NN, -1), itf.reshape(h, NN, -1), srcf,
            ts2[sl], neighbors_edges_timestamps[sl], vidx32[sl], *wargs))
    return jnp.concatenate(outs, axis=0)


# speedup vs baseline: 1.1535x; 1.1509x over previous
import functools
import math

import jax
import jax.numpy as jnp
from jax import lax
from jax.experimental import pallas as pl
from jax.experimental.pallas import tpu as pltpu
from jax.experimental.pallas import tpu_sc as plsc


def _mm(a, b):
    return lax.dot_general(a, b, (((1,), (0,)), ((), ())),
                           preferred_element_type=jnp.float32)


def _mm_t(a, b):
    return lax.dot_general(a, b, (((1,), (1,)), ((), ())),
                           preferred_element_type=jnp.float32)


def _bf16_bits(x):
    xr = x.astype(jnp.bfloat16).astype(jnp.float32)
    return lax.bitcast_convert_type(xr, jnp.uint32)


def _prep(uf, um, isf_t, imem_t):
    n, ud = uf.shape
    fd = isf_t.shape[1]
    md = imem_t.shape[1]
    hd = ud // 2
    blk = 2000

    def body(uf_ref, um_ref, isf_ref, imem_ref, u_ref, it_ref):
        a = uf_ref[...] + um_ref[...]
        lo = _bf16_bits(a[:, :hd]) >> 16
        hi = _bf16_bits(a[:, hd:]) & jnp.uint32(0xFFFF0000)
        u_ref[...] = lax.bitcast_convert_type(lo | hi, jnp.int32)
        ilo = _bf16_bits(isf_ref[...]) >> 16
        ihi = _bf16_bits(imem_ref[...]) & jnp.uint32(0xFFFF0000)
        it_ref[...] = lax.bitcast_convert_type(ilo | ihi, jnp.int32)

    assert n % blk == 0 and fd == md == hd
    return pl.pallas_call(
        body,
        grid=(n // blk,),
        in_specs=[pl.BlockSpec((blk, ud), lambda i: (i, 0)),
                  pl.BlockSpec((blk, ud), lambda i: (i, 0)),
                  pl.BlockSpec((blk, fd), lambda i: (i, 0)),
                  pl.BlockSpec((blk, md), lambda i: (i, 0))],
        out_specs=[pl.BlockSpec((blk, hd), lambda i: (i, 0)),
                   pl.BlockSpec((blk, hd), lambda i: (i, 0))],
        out_shape=[jax.ShapeDtypeStruct((n, hd), jnp.int32),
                   jax.ShapeDtypeStruct((n, hd), jnp.int32)],
    )(uf, um, isf_t, imem_t)


def _sc_gather(U, IT, vidx, iidx, inv):
    info = plsc.get_sparse_core_info()
    NC, NS = info.num_cores, info.num_subcores
    NW = NC * NS
    E = vidx.shape[0]
    Bq = inv.shape[0]
    UD = U.shape[1]
    KD2 = IT.shape[1]
    C = 128
    per_w = E // NW
    n_chunks = per_w // C
    src_per_w = Bq // NW
    assert per_w * NW == E and n_chunks * C == per_w and n_chunks % 2 == 0
    assert src_per_w * NW == Bq and src_per_w <= C

    mesh = plsc.VectorSubcoreMesh(core_axis_name="c", subcore_axis_name="s")

    @functools.partial(
        pl.kernel,
        mesh=mesh,
        out_type=[jax.ShapeDtypeStruct((E, UD), jnp.int32),
                  jax.ShapeDtypeStruct((E, KD2), jnp.int32),
                  jax.ShapeDtypeStruct((Bq, UD), jnp.int32)],
        scratch_types=[pltpu.VMEM((per_w,), jnp.int32),
                       pltpu.VMEM((per_w,), jnp.int32),
                       pltpu.VMEM((C, UD), jnp.int32),
                       pltpu.VMEM((C, KD2), jnp.int32),
                       pltpu.VMEM((C, UD), jnp.int32),
                       pltpu.VMEM((C, KD2), jnp.int32),
                       pltpu.SemaphoreType.DMA,
                       pltpu.SemaphoreType.DMA],
    )
    def gather_k(u_hbm, it_hbm, vidx_hbm, iidx_hbm, inv_hbm,
                 nb_out, it_out, src_out,
                 idx_u, idx_i, nb_a, it_a, nb_b, it_b, sem_a, sem_b):
        wid = lax.axis_index("s") * NC + lax.axis_index("c")
        base0 = wid * per_w

        pltpu.sync_copy(vidx_hbm.at[pl.ds(base0, per_w)], idx_u)
        pltpu.sync_copy(iidx_hbm.at[pl.ds(base0, per_w)], idx_i)

        def startg(j, nb_v, it_v, sem):
            off = j * C
            c1 = pltpu.async_copy(u_hbm.at[idx_u.at[pl.ds(off, C)]], nb_v, sem)
            c2 = pltpu.async_copy(it_hbm.at[idx_i.at[pl.ds(off, C)]], it_v, sem)
            return c1, c2

        pa = startg(0, nb_a, it_a, sem_a)
        pb = startg(1, nb_b, it_b, sem_b)

        def body(t, carry):
            j0 = 2 * t
            base = base0 + j0 * C
            pltpu.make_async_copy(u_hbm.at[idx_u.at[pl.ds(0, C)]], nb_a,
                                  sem_a).wait()
            pltpu.make_async_copy(it_hbm.at[idx_i.at[pl.ds(0, C)]], it_a,
                                  sem_a).wait()
            pltpu.sync_copy(nb_a, nb_out.at[pl.ds(base, C)])
            pltpu.sync_copy(it_a, it_out.at[pl.ds(base, C)])

            @pl.when(t + 1 < n_chunks // 2)
            def _():
                startg(j0 + 2, nb_a, it_a, sem_a)

            pltpu.make_async_copy(u_hbm.at[idx_u.at[pl.ds(0, C)]], nb_b,
                                  sem_b).wait()
            pltpu.make_async_copy(it_hbm.at[idx_i.at[pl.ds(0, C)]], it_b,
                                  sem_b).wait()
            pltpu.sync_copy(nb_b, nb_out.at[pl.ds(base + C, C)])
            pltpu.sync_copy(it_b, it_out.at[pl.ds(base + C, C)])

            @pl.when(t + 1 < n_chunks // 2)
            def _():
                startg(j0 + 3, nb_b, it_b, sem_b)

            return carry

        lax.fori_loop(0, n_chunks // 2, body, 0)

        sbase = wid * src_per_w
        pltpu.sync_copy(inv_hbm.at[pl.ds(sbase, src_per_w)],
                        idx_u.at[pl.ds(0, src_per_w)])
        pltpu.async_copy(u_hbm.at[idx_u.at[pl.ds(0, src_per_w)]],
                         nb_a.at[pl.ds(0, src_per_w)], sem_a).wait()
        pltpu.sync_copy(nb_a.at[pl.ds(0, src_per_w)],
                        src_out.at[pl.ds(sbase, src_per_w)])

    return gather_k(U, IT, vidx, iidx, inv)


def _unpack(p):
    lo = lax.bitcast_convert_type(jnp.left_shift(p, 16), jnp.float32)
    hi = lax.bitcast_convert_type(
        jnp.bitwise_and(p, jnp.int32(-65536)), jnp.float32)
    return lo, hi


def _fast_cos(x):
    n = jnp.round(x * jnp.float32(0.15915494309189535))
    r = x - n * jnp.float32(6.2831854820251465)
    r = r - n * jnp.float32(-1.7484556000744487e-07)
    r2 = r * r
    p = jnp.float32(-1.147074559772972e-11)
    p = p * r2 + jnp.float32(2.08767569878681e-09)
    p = p * r2 + jnp.float32(-2.7557319223985893e-07)
    p = p * r2 + jnp.float32(2.480158730158730e-05)
    p = p * r2 + jnp.float32(-1.3888888888888889e-03)
    p = p * r2 + jnp.float32(4.1666666666666664e-02)
    p = p * r2 + jnp.float32(-0.5)
    return p * r2 + jnp.float32(1.0)


def _main_body(nb_ref, it_ref, src_ref, ts_ref, ets_ref, vidx_ref,
               tw_ref, tb_ref, wq_ref, bq_ref, wk_ref, bk_ref, wv_ref, bv_ref,
               wo_ref, bo_ref, w1_ref, b1_ref, w2_ref, b2_ref, o_ref):
    HD = src_ref.shape[1]
    UD = 2 * HD
    TD = tw_ref.shape[1]
    QD = wq_ref.shape[0]
    H = 2
    DH = QD // H

    src_lo, src_hi = _unpack(src_ref[...])
    tw = tw_ref[...]
    tb = tb_ref[...]

    st = jnp.cos(tb)
    wq = wq_ref[...]
    q = (_mm_t(src_lo, wq[:, :HD]) + _mm_t(src_hi, wq[:, HD:UD])
         + _mm_t(st, wq[:, UD:]) + bq_ref[...])
    q = q * jnp.float32(1.0 / math.sqrt(DH))

    wk = wk_ref[...]
    g0 = _mm(q[:, :DH], wk[:DH, :])
    g1 = _mm(q[:, DH:], wk[DH:, :])
    bk = bk_ref[...]
    sb0 = jnp.sum(q[:, :DH] * bk[:, :DH], axis=-1, keepdims=True)
    sb1 = jnp.sum(q[:, DH:] * bk[:, DH:], axis=-1, keepdims=True)

    deltas = ts_ref[...] - ets_ref[...]
    TDr = tw.reshape(1, 1, TD)
    TBr = tb.reshape(1, 1, TD)
    ete = _fast_cos(deltas[:, :, None] * TDr + TBr)

    nb_lo, nb_hi = _unpack(nb_ref[...])
    it_lo, it_hi = _unpack(it_ref[...])

    mask = vidx_ref[...] == 0
    neg = jnp.float32(-1e10)

    o1 = UD
    o2 = UD + TD
    o3 = UD + TD + HD

    BB = mask.shape[0]
    NN = mask.shape[1]

    NG = 32
    ngroups = BB // NG
    i_b = lax.broadcasted_iota(jnp.int32, (NG, 1, 2 * NG), 0)
    i_j = lax.broadcasted_iota(jnp.int32, (NG, 1, 2 * NG), 2)
    E0 = (i_j == i_b).astype(jnp.float32)
    E1 = (i_j == i_b + NG).astype(jnp.float32)

    sp0, sp1 = [], []
    for gi in range(ngroups):
        sl = slice(gi * NG, (gi + 1) * NG)
        Gg = jnp.concatenate([g0[sl], g1[sl]], axis=0)
        M = (_mm_t(nb_lo[sl].reshape(NG * NN, HD), Gg[:, :HD])
             + _mm_t(nb_hi[sl].reshape(NG * NN, HD), Gg[:, HD:o1])
             + _mm_t(ete[sl].reshape(NG * NN, TD), Gg[:, o1:o2])
             + _mm_t(it_lo[sl].reshape(NG * NN, HD), Gg[:, o2:o3])
             + _mm_t(it_hi[sl].reshape(NG * NN, HD), Gg[:, o3:]))
        R = M.reshape(NG, NN, 2 * NG)
        sp0.append(jnp.sum(R * E0, axis=-1))
        sp1.append(jnp.sum(R * E1, axis=-1))
    s0 = jnp.concatenate(sp0, axis=0) + sb0
    s1 = jnp.concatenate(sp1, axis=0) + sb1

    def soft(s):
        s = jnp.where(mask, neg, s)
        m = jnp.max(s, axis=-1, keepdims=True)
        e = jnp.exp(s - m)
        return e / jnp.sum(e, axis=-1, keepdims=True)

    a0 = soft(s0)
    a1 = soft(s1)

    wv = wv_ref[...]
    bv = bv_ref[...]

    def head_ctx(a, lo):
        w = wv[lo:lo + DH, :]
        a3 = a[:, :, None]
        pooled = jnp.concatenate(
            [jnp.sum(a3 * nb_lo, axis=1), jnp.sum(a3 * nb_hi, axis=1),
             jnp.sum(a3 * ete, axis=1), jnp.sum(a3 * it_lo, axis=1),
             jnp.sum(a3 * it_hi, axis=1)], axis=-1)
        return _mm_t(pooled, w) + bv[:, lo:lo + DH]

    ctx = jnp.concatenate([head_ctx(a0, 0), head_ctx(a1, DH)], axis=-1)
    attn_out = _mm_t(ctx, wo_ref[...]) + bo_ref[...]
    allm = jnp.all(mask, axis=-1, keepdims=True)
    attn_out = jnp.where(allm, jnp.float32(0.0), attn_out)

    w1 = w1_ref[...]
    hid = jnp.maximum(
        _mm_t(attn_out, w1[:, :QD]) + _mm_t(src_lo, w1[:, QD:QD + HD])
        + _mm_t(src_hi, w1[:, QD + HD:]) + b1_ref[...],
        jnp.float32(0.0))
    o_ref[...] = _mm_t(hid, w2_ref[...]) + b2_ref[...]


def _main(nb, it, src, ts, ets, vidx,
          tw, tb, Wq, bq, Wk, bk, Wv, bv, Wo, bo, W1, b1, W2, b2):
    B, NN, HD = nb.shape
    KD2 = it.shape[2]
    OD = W2.shape[0]
    BB = 256
    assert B % BB == 0
    grid = (B // BB,)

    def blk(shape):
        nd = len(shape)
        return pl.BlockSpec(shape, lambda i, _n=nd: (i,) + (0,) * (_n - 1))

    def full(a):
        nd = a.ndim
        return pl.BlockSpec(a.shape, lambda i, _n=nd: (0,) * _n)

    weights = (tw, tb, Wq, bq, Wk, bk, Wv, bv, Wo, bo, W1, b1, W2, b2)
    in_specs = [blk((BB, NN, HD)), blk((BB, NN, KD2)),
                blk((BB, HD)), blk((BB, 1)), blk((BB, NN)), blk((BB, NN))]
    in_specs += [full(w) for w in weights]

    return pl.pallas_call(
        _main_body,
        grid=grid,
        in_specs=in_specs,
        out_specs=blk((BB, OD)),
        out_shape=jax.ShapeDtypeStruct((B, OD), jnp.float32),
    )(nb, it, src, ts, ets, vidx, *weights)


def kernel(inviters_idxs, timestamps, num_layers, num_neighbors,
           neighbors_voters_idxs, neighbors_items_idxs,
           neighbors_edges_timestamps, users_features, users_memory,
           items_static_features, items_memory, time_w, time_b,
           Wq, bq, Wk, bk, Wv, bv, Wo, bo, W1, b1, W2, b2):
    B = inviters_idxs.shape[0]
    NN = neighbors_voters_idxs.shape[1]

    U, IT = _prep(users_features, users_memory,
                  items_static_features, items_memory)

    vidx32 = neighbors_voters_idxs.astype(jnp.int32)
    iidx32 = neighbors_items_idxs.astype(jnp.int32)
    inv = inviters_idxs.astype(jnp.int32)
    ts2 = timestamps.reshape(B, 1)
    wargs = (time_w.reshape(1, -1), time_b.reshape(1, -1),
             Wq, bq.reshape(1, -1), Wk, bk.reshape(1, -1),
             Wv, bv.reshape(1, -1), Wo, bo.reshape(1, -1),
             W1, b1.reshape(1, -1), W2, b2.reshape(1, -1))

    nparts = 2
    h = B // nparts
    outs = []
    for p in range(nparts):
        sl = slice(p * h, (p + 1) * h)
        nbf, itf, srcf = _sc_gather(
            U, IT, vidx32[sl].reshape(-1), iidx32[sl].reshape(-1), inv[sl])
        outs.append(_main(
            nbf.reshape(h, NN, -1), itf.reshape(h, NN, -1), srcf,
            ts2[sl], neighbors_edges_timestamps[sl], vidx32[sl], *wargs))
    return jnp.concatenate(outs, axis=0)


# speedup vs baseline: 1.3207x; 1.1450x over previous
import functools
import math

import jax
import jax.numpy as jnp
from jax import lax
from jax.experimental import pallas as pl
from jax.experimental.pallas import tpu as pltpu
from jax.experimental.pallas import tpu_sc as plsc


def _mm(a, b):
    return lax.dot_general(a, b, (((1,), (0,)), ((), ())),
                           preferred_element_type=jnp.float32)


def _mm_t(a, b):
    return lax.dot_general(a, b, (((1,), (1,)), ((), ())),
                           preferred_element_type=jnp.float32)


def _bf16_bits(x):
    xr = x.astype(jnp.bfloat16).astype(jnp.float32)
    return lax.bitcast_convert_type(xr, jnp.uint32)


def _prep(uf, um, isf_t, imem_t):
    n, ud = uf.shape
    fd = isf_t.shape[1]
    md = imem_t.shape[1]
    hd = ud // 2
    blk = 2000

    def body(uf_ref, um_ref, isf_ref, imem_ref, u_ref, it_ref):
        a = uf_ref[...] + um_ref[...]
        lo = _bf16_bits(a[:, :hd]) >> 16
        hi = _bf16_bits(a[:, hd:]) & jnp.uint32(0xFFFF0000)
        u_ref[...] = lax.bitcast_convert_type(lo | hi, jnp.int32)
        ilo = _bf16_bits(isf_ref[...]) >> 16
        ihi = _bf16_bits(imem_ref[...]) & jnp.uint32(0xFFFF0000)
        it_ref[...] = lax.bitcast_convert_type(ilo | ihi, jnp.int32)

    assert n % blk == 0 and fd == md == hd
    return pl.pallas_call(
        body,
        grid=(n // blk,),
        in_specs=[pl.BlockSpec((blk, ud), lambda i: (i, 0)),
                  pl.BlockSpec((blk, ud), lambda i: (i, 0)),
                  pl.BlockSpec((blk, fd), lambda i: (i, 0)),
                  pl.BlockSpec((blk, md), lambda i: (i, 0))],
        out_specs=[pl.BlockSpec((blk, hd), lambda i: (i, 0)),
                   pl.BlockSpec((blk, hd), lambda i: (i, 0))],
        out_shape=[jax.ShapeDtypeStruct((n, hd), jnp.int32),
                   jax.ShapeDtypeStruct((n, hd), jnp.int32)],
    )(uf, um, isf_t, imem_t)


def _sc_gather(U, IT, vidx, iidx, inv):
    info = plsc.get_sparse_core_info()
    NC, NS = info.num_cores, info.num_subcores
    NW = NC * NS
    E = vidx.shape[0]
    Bq = inv.shape[0]
    UD = U.shape[1]
    KD2 = IT.shape[1]
    C = 128
    per_w = E // NW
    n_chunks = per_w // C
    src_per_w = Bq // NW
    assert per_w * NW == E and n_chunks * C == per_w and n_chunks % 2 == 0
    assert src_per_w * NW == Bq and src_per_w <= C

    mesh = plsc.VectorSubcoreMesh(core_axis_name="c", subcore_axis_name="s")

    @functools.partial(
        pl.kernel,
        mesh=mesh,
        out_type=[jax.ShapeDtypeStruct((E, UD), jnp.int32),
                  jax.ShapeDtypeStruct((E, KD2), jnp.int32),
                  jax.ShapeDtypeStruct((Bq, UD), jnp.int32)],
        scratch_types=[pltpu.VMEM((per_w,), jnp.int32),
                       pltpu.VMEM((per_w,), jnp.int32),
                       pltpu.VMEM((C, UD), jnp.int32),
                       pltpu.VMEM((C, KD2), jnp.int32),
                       pltpu.VMEM((C, UD), jnp.int32),
                       pltpu.VMEM((C, KD2), jnp.int32),
                       pltpu.SemaphoreType.DMA,
                       pltpu.SemaphoreType.DMA],
    )
    def gather_k(u_hbm, it_hbm, vidx_hbm, iidx_hbm, inv_hbm,
                 nb_out, it_out, src_out,
                 idx_u, idx_i, nb_a, it_a, nb_b, it_b, sem_a, sem_b):
        wid = lax.axis_index("s") * NC + lax.axis_index("c")
        base0 = wid * per_w

        pltpu.sync_copy(vidx_hbm.at[pl.ds(base0, per_w)], idx_u)
        pltpu.sync_copy(iidx_hbm.at[pl.ds(base0, per_w)], idx_i)

        def startg(j, nb_v, it_v, sem):
            off = j * C
            c1 = pltpu.async_copy(u_hbm.at[idx_u.at[pl.ds(off, C)]], nb_v, sem)
            c2 = pltpu.async_copy(it_hbm.at[idx_i.at[pl.ds(off, C)]], it_v, sem)
            return c1, c2

        pa = startg(0, nb_a, it_a, sem_a)
        pb = startg(1, nb_b, it_b, sem_b)

        def body(t, carry):
            j0 = 2 * t
            base = base0 + j0 * C
            pltpu.make_async_copy(u_hbm.at[idx_u.at[pl.ds(0, C)]], nb_a,
                                  sem_a).wait()
            pltpu.make_async_copy(it_hbm.at[idx_i.at[pl.ds(0, C)]], it_a,
                                  sem_a).wait()
            pltpu.sync_copy(nb_a, nb_out.at[pl.ds(base, C)])
            pltpu.sync_copy(it_a, it_out.at[pl.ds(base, C)])

            @pl.when(t + 1 < n_chunks // 2)
            def _():
                startg(j0 + 2, nb_a, it_a, sem_a)

            pltpu.make_async_copy(u_hbm.at[idx_u.at[pl.ds(0, C)]], nb_b,
                                  sem_b).wait()
            pltpu.make_async_copy(it_hbm.at[idx_i.at[pl.ds(0, C)]], it_b,
                                  sem_b).wait()
            pltpu.sync_copy(nb_b, nb_out.at[pl.ds(base + C, C)])
            pltpu.sync_copy(it_b, it_out.at[pl.ds(base + C, C)])

            @pl.when(t + 1 < n_chunks // 2)
            def _():
                startg(j0 + 3, nb_b, it_b, sem_b)

            return carry

        lax.fori_loop(0, n_chunks // 2, body, 0)

        sbase = wid * src_per_w
        pltpu.sync_copy(inv_hbm.at[pl.ds(sbase, src_per_w)],
                        idx_u.at[pl.ds(0, src_per_w)])
        pltpu.async_copy(u_hbm.at[idx_u.at[pl.ds(0, src_per_w)]],
                         nb_a.at[pl.ds(0, src_per_w)], sem_a).wait()
        pltpu.sync_copy(nb_a.at[pl.ds(0, src_per_w)],
                        src_out.at[pl.ds(sbase, src_per_w)])

    return gather_k(U, IT, vidx, iidx, inv)


def _unpack(p):
    lo = lax.bitcast_convert_type(jnp.left_shift(p, 16), jnp.float32)
    hi = lax.bitcast_convert_type(
        jnp.bitwise_and(p, jnp.int32(-65536)), jnp.float32)
    return lo, hi


def _fast_cos(x):
    n = jnp.round(x * jnp.float32(0.15915494309189535))
    r = x - n * jnp.float32(6.2831854820251465)
    r = r - n * jnp.float32(-1.7484556000744487e-07)
    r2 = r * r
    p = jnp.float32(-1.147074559772972e-11)
    p = p * r2 + jnp.float32(2.08767569878681e-09)
    p = p * r2 + jnp.float32(-2.7557319223985893e-07)
    p = p * r2 + jnp.float32(2.480158730158730e-05)
    p = p * r2 + jnp.float32(-1.3888888888888889e-03)
    p = p * r2 + jnp.float32(4.1666666666666664e-02)
    p = p * r2 + jnp.float32(-0.5)
    return p * r2 + jnp.float32(1.0)


def _main_body(nb_ref, it_ref, src_ref, ts_ref, ets_ref, vidx_ref,
               tw_ref, tb_ref, wq_ref, bq_ref, wk_ref, bk_ref, wv_ref, bv_ref,
               wo_ref, bo_ref, w1_ref, b1_ref, w2_ref, b2_ref, o_ref):
    HD = src_ref.shape[1]
    UD = 2 * HD
    TD = tw_ref.shape[1]
    QD = wq_ref.shape[0]
    H = 2
    DH = QD // H

    src_lo, src_hi = _unpack(src_ref[...])
    tw = tw_ref[...]
    tb = tb_ref[...]

    st = jnp.cos(tb)
    wq = wq_ref[...]
    q = (_mm_t(src_lo, wq[:, :HD]) + _mm_t(src_hi, wq[:, HD:UD])
         + _mm_t(st, wq[:, UD:]) + bq_ref[...])
    q = q * jnp.float32(1.0 / math.sqrt(DH))

    wk = wk_ref[...]
    g0 = _mm(q[:, :DH], wk[:DH, :])
    g1 = _mm(q[:, DH:], wk[DH:, :])
    bk = bk_ref[...]
    sb0 = jnp.sum(q[:, :DH] * bk[:, :DH], axis=-1, keepdims=True)
    sb1 = jnp.sum(q[:, DH:] * bk[:, DH:], axis=-1, keepdims=True)

    deltas = ts_ref[...] - ets_ref[...]
    TDr = tw.reshape(1, 1, TD)
    TBr = tb.reshape(1, 1, TD)
    ete = _fast_cos(deltas[:, :, None] * TDr + TBr)

    nb_lo, nb_hi = _unpack(nb_ref[...])
    it_lo, it_hi = _unpack(it_ref[...])

    mask = vidx_ref[...] == 0
    neg = jnp.float32(-1e10)

    o1 = UD
    o2 = UD + TD
    o3 = UD + TD + HD

    BB = mask.shape[0]
    NN = mask.shape[1]

    NG = 32
    ngroups = BB // NG
    i_b = lax.broadcasted_iota(jnp.int32, (NG, 1, 2 * NG), 0)
    i_j = lax.broadcasted_iota(jnp.int32, (NG, 1, 2 * NG), 2)
    E0 = (i_j == i_b).astype(jnp.float32)
    E1 = (i_j == i_b + NG).astype(jnp.float32)

    sp0, sp1 = [], []
    for gi in range(ngroups):
        sl = slice(gi * NG, (gi + 1) * NG)
        Gg = jnp.concatenate([g0[sl], g1[sl]], axis=0)
        M = (_mm_t(nb_lo[sl].reshape(NG * NN, HD), Gg[:, :HD])
             + _mm_t(nb_hi[sl].reshape(NG * NN, HD), Gg[:, HD:o1])
             + _mm_t(ete[sl].reshape(NG * NN, TD), Gg[:, o1:o2])
             + _mm_t(it_lo[sl].reshape(NG * NN, HD), Gg[:, o2:o3])
             + _mm_t(it_hi[sl].reshape(NG * NN, HD), Gg[:, o3:]))
        R = M.reshape(NG, NN, 2 * NG)
        sp0.append(jnp.sum(R * E0, axis=-1))
        sp1.append(jnp.sum(R * E1, axis=-1))
    s0 = jnp.concatenate(sp0, axis=0) + sb0
    s1 = jnp.concatenate(sp1, axis=0) + sb1

    def soft(s):
        s = jnp.where(mask, neg, s)
        m = jnp.max(s, axis=-1, keepdims=True)
        e = jnp.exp(s - m)
        return e / jnp.sum(e, axis=-1, keepdims=True)

    a0 = soft(s0)
    a1 = soft(s1)

    wv = wv_ref[...]
    bv = bv_ref[...]

    bd_mask = ((lax.broadcasted_iota(jnp.int32, (2 * NG, NG * NN), 1) // NN)
               == (lax.broadcasted_iota(jnp.int32, (2 * NG, NG * NN), 0)
                   % NG)).astype(jnp.float32)
    p0p, p1p = [], []
    for gi in range(ngroups):
        sl = slice(gi * NG, (gi + 1) * NG)
        a_cat = jnp.concatenate([a0[sl], a1[sl]], axis=0)
        a_bd = jnp.tile(a_cat, (1, NG)) * bd_mask
        pooled = jnp.concatenate(
            [_mm(a_bd, nb_lo[sl].reshape(NG * NN, HD)),
             _mm(a_bd, nb_hi[sl].reshape(NG * NN, HD)),
             _mm(a_bd, ete[sl].reshape(NG * NN, TD)),
             _mm(a_bd, it_lo[sl].reshape(NG * NN, HD)),
             _mm(a_bd, it_hi[sl].reshape(NG * NN, HD))], axis=-1)
        p0p.append(pooled[:NG])
        p1p.append(pooled[NG:])
    pooled0 = jnp.concatenate(p0p, axis=0)
    pooled1 = jnp.concatenate(p1p, axis=0)
    ctx = jnp.concatenate(
        [_mm_t(pooled0, wv[:DH, :]) + bv[:, :DH],
         _mm_t(pooled1, wv[DH:, :]) + bv[:, DH:]], axis=-1)
    attn_out = _mm_t(ctx, wo_ref[...]) + bo_ref[...]
    allm = jnp.all(mask, axis=-1, keepdims=True)
    attn_out = jnp.where(allm, jnp.float32(0.0), attn_out)

    w1 = w1_ref[...]
    hid = jnp.maximum(
        _mm_t(attn_out, w1[:, :QD]) + _mm_t(src_lo, w1[:, QD:QD + HD])
        + _mm_t(src_hi, w1[:, QD + HD:]) + b1_ref[...],
        jnp.float32(0.0))
    o_ref[...] = _mm_t(hid, w2_ref[...]) + b2_ref[...]


def _main(nb, it, src, ts, ets, vidx,
          tw, tb, Wq, bq, Wk, bk, Wv, bv, Wo, bo, W1, b1, W2, b2):
    B, NN, HD = nb.shape
    KD2 = it.shape[2]
    OD = W2.shape[0]
    BB = 256
    assert B % BB == 0
    grid = (B // BB,)

    def blk(shape):
        nd = len(shape)
        return pl.BlockSpec(shape, lambda i, _n=nd: (i,) + (0,) * (_n - 1))

    def full(a):
        nd = a.ndim
        return pl.BlockSpec(a.shape, lambda i, _n=nd: (0,) * _n)

    weights = (tw, tb, Wq, bq, Wk, bk, Wv, bv, Wo, bo, W1, b1, W2, b2)
    in_specs = [blk((BB, NN, HD)), blk((BB, NN, KD2)),
                blk((BB, HD)), blk((BB, 1)), blk((BB, NN)), blk((BB, NN))]
    in_specs += [full(w) for w in weights]

    return pl.pallas_call(
        _main_body,
        grid=grid,
        in_specs=in_specs,
        out_specs=blk((BB, OD)),
        out_shape=jax.ShapeDtypeStruct((B, OD), jnp.float32),
    )(nb, it, src, ts, ets, vidx, *weights)


def kernel(inviters_idxs, timestamps, num_layers, num_neighbors,
           neighbors_voters_idxs, neighbors_items_idxs,
           neighbors_edges_timestamps, users_features, users_memory,
           items_static_features, items_memory, time_w, time_b,
           Wq, bq, Wk, bk, Wv, bv, Wo, bo, W1, b1, W2, b2):
    B = inviters_idxs.shape[0]
    NN = neighbors_voters_idxs.shape[1]

    U, IT = _prep(users_features, users_memory,
                  items_static_features, items_memory)

    vidx32 = neighbors_voters_idxs.astype(jnp.int32)
    iidx32 = neighbors_items_idxs.astype(jnp.int32)
    inv = inviters_idxs.astype(jnp.int32)
    ts2 = timestamps.reshape(B, 1)
    wargs = (time_w.reshape(1, -1), time_b.reshape(1, -1),
             Wq, bq.reshape(1, -1), Wk, bk.reshape(1, -1),
             Wv, bv.reshape(1, -1), Wo, bo.reshape(1, -1),
             W1, b1.reshape(1, -1), W2, b2.reshape(1, -1))

    nparts = 2
    h = B // nparts
    outs = []
    for p in range(nparts):
        sl = slice(p * h, (p + 1) * h)
        nbf, itf, srcf = _sc_gather(
            U, IT, vidx32[sl].reshape(-1), iidx32[sl].reshape(-1), inv[sl])
        outs.append(_main(
            nbf.reshape(h, NN, -1), itf.reshape(h, NN, -1), srcf,
            ts2[sl], neighbors_edges_timestamps[sl], vidx32[sl], *wargs))
    return jnp.concatenate(outs, axis=0)
